# baseline (speedup 1.0000x reference)
import functools

import jax
import jax.numpy as jnp
from jax import lax
from jax.experimental import pallas as pl
from jax.experimental.pallas import tpu as pltpu
from jax.experimental.pallas import tpu_sc as plsc

N = 10000
NN = 32
IN_C = 128
OUT_C = 128
K = 15
KP_EXTENT = 0.1

NP = 10240
E = NP * NN

B = 320
G = B // 8
EPB = B * NN
NBLK = NP // B

NC = 2
NS = 16
NW = NC * NS
PER_W = E // NW
CH = 80
NCHUNK = PER_W // CH
NBUF = 4
NSUP = NCHUNK // NBUF


POSW = 16
CH8 = CH * POSW // 128


def _sc_gather(features, support_pad, idx_grp):
    mesh = plsc.VectorSubcoreMesh(core_axis_name="c", subcore_axis_name="s")

    @functools.partial(
        pl.kernel,
        mesh=mesh,
        out_type=[
            jax.ShapeDtypeStruct((E, IN_C), jnp.float32),
            jax.ShapeDtypeStruct((E, POSW), jnp.float32),
        ],
        scratch_types=(
            [pltpu.VMEM((NCHUNK, CH), jnp.int32)]
            + [pltpu.VMEM((CH, IN_C), jnp.float32) for _ in range(NBUF)]
            + [pltpu.VMEM((CH, POSW), jnp.float32) for _ in range(NBUF)]
            + [pltpu.SemaphoreType.DMA] * (2 * NBUF)
        ),
        compiler_params=pltpu.CompilerParams(use_tc_tiling_on_sc=False),
    )
    def gather_kernel(feat_hbm, pos_hbm, idx_hbm,
                      nf_hbm, npos_hbm, idx_all, *bufs):
        feat_v = bufs[:NBUF]
        pos_v = bufs[NBUF:2 * NBUF]
        gsem = bufs[2 * NBUF:3 * NBUF]
        wsem = bufs[3 * NBUF:]
        wid = lax.axis_index("s") * NC + lax.axis_index("c")
        base = wid * PER_W

        pltpu.sync_copy(idx_hbm.at[wid], idx_all)

        def start_g(c, j):
            pltpu.async_copy(feat_hbm.at[idx_all.at[c]], feat_v[j], gsem[j])
            pltpu.async_copy(pos_hbm.at[idx_all.at[c]], pos_v[j], gsem[j])

        def wait_g(j):
            pltpu.make_async_copy(feat_hbm.at[pl.ds(0, CH)], feat_v[j],
                                  gsem[j]).wait()
            pltpu.make_async_copy(pos_hbm.at[pl.ds(0, CH)], pos_v[j],
                                  gsem[j]).wait()

        def start_w(c, j):
            pltpu.async_copy(feat_v[j], nf_hbm.at[pl.ds(base + c * CH, CH)],
                             wsem[j])
            pltpu.async_copy(pos_v[j],
                             npos_hbm.at[pl.ds(base + c * CH, CH)],
                             wsem[j])

        def wait_w(j):
            pltpu.make_async_copy(feat_v[j], nf_hbm.at[pl.ds(0, CH)],
                                  wsem[j]).wait()
            pltpu.make_async_copy(pos_v[j],
                                  npos_hbm.at[pl.ds(0, CH)],
                                  wsem[j]).wait()

        for j in range(NBUF):
            start_g(j, j)

        def body(cc, carry):
            for j in range(NBUF):
                c = cc * NBUF + j
                wait_g(j)
                start_w(c, j)

                @pl.when(cc < NSUP - 1)
                def _():
                    wait_w(j)
                    start_g(c + NBUF, j)

            return carry

        lax.fori_loop(0, NSUP, body, 0)
        for j in range(NBUF):
            wait_w(j)

    return gather_kernel(features, support_pad, idx_grp)


def _tc_body(nf_ref, npos_hbm, q_ref, kpt_ref, kv_ref, out_ref,
             wf_ref, npos_v, psem):
    f32 = jnp.float32
    bf16 = jnp.bfloat16
    i = pl.program_id(0)

    cp = pltpu.make_async_copy(npos_hbm.at[pl.ds(i * EPB, EPB)], npos_v, psem)
    cp.start()
    cp.wait()

    q = q_ref[...]
    qe = jnp.broadcast_to(q[:, None, :], (B, NN, 4)).reshape(EPB, 4)
    npos = npos_v[...]
    sq = jnp.zeros((EPB, 16), f32)
    for d in range(3):
        dd = (npos[:, d:d + 1] - qe[:, d:d + 1]) - kpt_ref[d:d + 1, :]
        sq = sq + dd * dd
    w_t = jnp.maximum(1.0 - jnp.sqrt(sq) * (1.0 / KP_EXTENT), 0.0)

    ri = lax.broadcasted_iota(jnp.int32, (128, 2 * 128), 0)
    ci = lax.broadcasted_iota(jnp.int32, (128, 2 * 128), 1)
    mask = ((ri % 8) == (ci // NN)).astype(f32)
    ri2 = lax.broadcasted_iota(jnp.int32, (128, 16), 0)
    ki = lax.broadcasted_iota(jnp.int32, (128, 16), 1)
    ksel = (ki == (ri2 // 8)).astype(f32)

    for g in range(G):
        w_g = w_t[256 * g:256 * (g + 1), :]
        bdw = lax.dot_general(ksel, w_g, (((1,), (1,)), ((), ())),
                              preferred_element_type=f32)
        bd = bdw * mask
        nf_g = nf_ref[pl.ds(256 * g, 256), :]
        wf_g = jnp.dot(bd, nf_g, preferred_element_type=f32)
        wf_ref[:, 8 * g:8 * (g + 1), :] = wf_g.astype(bf16).reshape(16, 8, 128)

    acc = jnp.zeros((B, OUT_C), f32)
    for k in range(16):
        acc = acc + jnp.dot(wf_ref[k], kv_ref[k], preferred_element_type=f32)
    out_ref[...] = acc


def kernel(query, support, edge_indices, features, K_points, K_values):
    f32 = jnp.float32
    idx_pad = jnp.pad(edge_indices.astype(jnp.int32).reshape(-1),
                      (0, E - N * NN))
    idx_grp = idx_pad.reshape(NW, NCHUNK, CH)
    support_pad = jnp.pad(support.astype(f32), ((0, 0), (0, POSW - 3)))
    nf, npos = _sc_gather(features.astype(f32), support_pad, idx_grp)

    q = jnp.pad(query.astype(f32), ((0, NP - N), (0, 1)))
    kpt = jnp.transpose(
        jnp.pad(K_points.astype(f32), ((0, 1), (0, 0)),
                constant_values=1e6))
    kpt = jnp.pad(kpt, ((0, 5), (0, 0)))
    kv = jnp.pad(K_values.astype(jnp.bfloat16),
                 ((0, 1), (0, 0), (0, 0)))

    out = pl.pallas_call(
        _tc_body,
        grid=(NBLK,),
        in_specs=[
            pl.BlockSpec((EPB, IN_C), lambda i: (i, 0)),
            pl.BlockSpec(memory_space=pl.ANY),
            pl.BlockSpec((B, 4), lambda i: (i, 0)),
            pl.BlockSpec((8, 16), lambda i: (0, 0)),
            pl.BlockSpec((16, 128, 128), lambda i: (0, 0, 0)),
        ],
        out_specs=pl.BlockSpec((B, OUT_C), lambda i: (i, 0)),
        out_shape=jax.ShapeDtypeStruct((NP, OUT_C), f32),
        scratch_shapes=[pltpu.VMEM((16, B, 128), jnp.bfloat16),
                        pltpu.VMEM((EPB, POSW), f32),
                        pltpu.SemaphoreType.DMA],
    )(nf, npos, q, kpt, kv)
    return out[:N]


# speedup vs baseline: 3.0037x; 3.0037x over previous
import functools

import jax
import jax.numpy as jnp
from jax import lax
from jax.experimental import pallas as pl
from jax.experimental.pallas import tpu as pltpu
from jax.experimental.pallas import tpu_sc as plsc

N = 10000
NN = 32
IN_C = 128
OUT_C = 128
K = 15
KP_EXTENT = 0.1

NP = 10240
E = NP * NN

B = 320
G = B // 8
EPB = B * NN
NBLK = NP // B

NC = 2
NS = 16
NW = NC * NS
PER_W = E // NW
CH = 80
NCHUNK = PER_W // CH
NBUF = 4
NSUP = NCHUNK // NBUF


def _sc_gather(features, supx, supy, supz, idx_grp):
    mesh = plsc.VectorSubcoreMesh(core_axis_name="c", subcore_axis_name="s")

    @functools.partial(
        pl.kernel,
        mesh=mesh,
        out_type=[
            jax.ShapeDtypeStruct((E, IN_C), jnp.float32),
            jax.ShapeDtypeStruct((E,), jnp.float32),
            jax.ShapeDtypeStruct((E,), jnp.float32),
            jax.ShapeDtypeStruct((E,), jnp.float32),
        ],
        scratch_types=(
            [pltpu.VMEM((NCHUNK, CH), jnp.int32)]
            + [pltpu.VMEM((CH, IN_C), jnp.float32) for _ in range(NBUF)]
            + [pltpu.VMEM((CH,), jnp.float32) for _ in range(3 * NBUF)]
            + [pltpu.SemaphoreType.DMA] * (2 * NBUF)
        ),
        compiler_params=pltpu.CompilerParams(use_tc_tiling_on_sc=False),
    )
    def gather_kernel(feat_hbm, sx_hbm, sy_hbm, sz_hbm, idx_hbm,
                      nf_hbm, xs_hbm, ys_hbm, zs_hbm, idx_all, *bufs):
        feat_v = bufs[:NBUF]
        pos_v = [bufs[NBUF + 3 * j:NBUF + 3 * (j + 1)] for j in range(NBUF)]
        gsem = bufs[4 * NBUF:5 * NBUF]
        wsem = bufs[5 * NBUF:]
        pos_hbms = (sx_hbm, sy_hbm, sz_hbm)
        out_hbms = (xs_hbm, ys_hbm, zs_hbm)
        wid = lax.axis_index("s") * NC + lax.axis_index("c")
        base = wid * PER_W

        pltpu.sync_copy(idx_hbm.at[wid], idx_all)

        def start_g(c, j):
            pltpu.async_copy(feat_hbm.at[idx_all.at[c]], feat_v[j], gsem[j])
            for t in range(3):
                pltpu.async_copy(pos_hbms[t].at[idx_all.at[c]],
                                 pos_v[j][t], gsem[j])

        def wait_g(j):
            pltpu.make_async_copy(feat_hbm.at[pl.ds(0, CH)], feat_v[j],
                                  gsem[j]).wait()
            for t in range(3):
                pltpu.make_async_copy(pos_hbms[t].at[pl.ds(0, CH)],
                                      pos_v[j][t], gsem[j]).wait()

        def start_w(c, j):
            off = base + c * CH
            pltpu.async_copy(feat_v[j], nf_hbm.at[pl.ds(off, CH)], wsem[j])
            for t in range(3):
                pltpu.async_copy(pos_v[j][t],
                                 out_hbms[t].at[pl.ds(off, CH)], wsem[j])

        def wait_w(j):
            pltpu.make_async_copy(feat_v[j], nf_hbm.at[pl.ds(0, CH)],
                                  wsem[j]).wait()
            for t in range(3):
                pltpu.make_async_copy(pos_v[j][t],
                                      out_hbms[t].at[pl.ds(0, CH)],
                                      wsem[j]).wait()

        for j in range(NBUF):
            start_g(j, j)

        def body(cc, carry):
            for j in range(NBUF):
                c = cc * NBUF + j
                wait_g(j)
                start_w(c, j)

                @pl.when(cc < NSUP - 1)
                def _():
                    wait_w(j)
                    start_g(c + NBUF, j)

            return carry

        lax.fori_loop(0, NSUP, body, 0)
        for j in range(NBUF):
            wait_w(j)

    return gather_kernel(features, supx, supy, supz, idx_grp)


def _tc_body(nf_ref, xs_ref, ys_ref, zs_ref, qx_ref, qy_ref, qz_ref,
             kpt_ref, kv_ref, out_ref, wf_ref):
    f32 = jnp.float32
    bf16 = jnp.bfloat16
    dx = (xs_ref[...] - qx_ref[...]).reshape(1, EPB)
    dy = (ys_ref[...] - qy_ref[...]).reshape(1, EPB)
    dz = (zs_ref[...] - qz_ref[...]).reshape(1, EPB)
    ddx = dx - kpt_ref[:, 0:1]
    ddy = dy - kpt_ref[:, 1:2]
    ddz = dz - kpt_ref[:, 2:3]
    sq = ddx * ddx + ddy * ddy + ddz * ddz
    w2 = jnp.maximum(1.0 - jnp.sqrt(sq) * (1.0 / KP_EXTENT), 0.0)

    ri = lax.broadcasted_iota(jnp.int32, (128, 2 * 128), 0)
    ci = lax.broadcasted_iota(jnp.int32, (128, 2 * 128), 1)
    mask = ((ri % 8) == (ci // NN)).astype(f32)

    for g in range(G):
        w_g = w2[:, 256 * g:256 * (g + 1)]
        bd = jnp.broadcast_to(w_g[:, None, :],
                              (16, 8, 256)).reshape(128, 256) * mask
        nf_g = nf_ref[pl.ds(256 * g, 256), :]
        wf_g = jnp.dot(bd, nf_g, preferred_element_type=f32)
        wf_ref[:, 8 * g:8 * (g + 1), :] = wf_g.astype(bf16).reshape(16, 8, 128)

    acc = jnp.zeros((B, OUT_C), f32)
    for k in range(16):
        acc = acc + jnp.dot(wf_ref[k], kv_ref[k], preferred_element_type=f32)
    out_ref[...] = acc


def kernel(query, support, edge_indices, features, K_points, K_values):
    f32 = jnp.float32
    padv = (jnp.arange(E - N * NN, dtype=jnp.int32) * 53) % N
    idx_pad = jnp.concatenate(
        [edge_indices.astype(jnp.int32).reshape(-1), padv])
    idx_grp = idx_pad.reshape(NW, NCHUNK, CH)
    sup = support.astype(f32)
    nf, xs, ys, zs = _sc_gather(features.astype(f32), sup[:, 0], sup[:, 1],
                                sup[:, 2], idx_grp)

    q = jnp.pad(query.astype(f32), ((0, NP - N), (0, 0)))
    qxr = jnp.repeat(q[:, 0], NN)
    qyr = jnp.repeat(q[:, 1], NN)
    qzr = jnp.repeat(q[:, 2], NN)
    kpt = jnp.pad(K_points.astype(f32), ((0, 1), (0, 1)),
                  constant_values=1e6)
    kv = jnp.pad(K_values.astype(jnp.bfloat16),
                 ((0, 1), (0, 0), (0, 0)))

    vec = pl.BlockSpec((EPB,), lambda i: (i,))
    out = pl.pallas_call(
        _tc_body,
        grid=(NBLK,),
        in_specs=[
            pl.BlockSpec((EPB, IN_C), lambda i: (i, 0)),
            vec, vec, vec, vec, vec, vec,
            pl.BlockSpec((16, 4), lambda i: (0, 0)),
            pl.BlockSpec((16, 128, 128), lambda i: (0, 0, 0)),
        ],
        out_specs=pl.BlockSpec((B, OUT_C), lambda i: (i, 0)),
        out_shape=jax.ShapeDtypeStruct((NP, OUT_C), f32),
        scratch_shapes=[pltpu.VMEM((16, B, 128), jnp.bfloat16)],
    )(nf, xs, ys, zs, qxr, qyr, qzr, kpt, kv)
    return out[:N]


# speedup vs baseline: 3.0090x; 1.0017x over previous
import functools

import jax
import jax.numpy as jnp
from jax import lax
from jax.experimental import pallas as pl
from jax.experimental.pallas import tpu as pltpu
from jax.experimental.pallas import tpu_sc as plsc

N = 10000
NN = 32
IN_C = 128
OUT_C = 128
K = 15
KP_EXTENT = 0.1

NP = 10240
E = NP * NN

B = 320
G = B // 8
EPB = B * NN
NBLK = NP // B

NC = 2
NS = 16
NW = NC * NS
PER_W = E // NW
CH = 128
NCHUNK = PER_W // CH
NBUF = 5
NSUP = NCHUNK // NBUF


def _sc_gather(features, supx, supy, supz, idx_grp):
    mesh = plsc.VectorSubcoreMesh(core_axis_name="c", subcore_axis_name="s")

    @functools.partial(
        pl.kernel,
        mesh=mesh,
        out_type=[
            jax.ShapeDtypeStruct((E, IN_C), jnp.float32),
            jax.ShapeDtypeStruct((E,), jnp.float32),
            jax.ShapeDtypeStruct((E,), jnp.float32),
            jax.ShapeDtypeStruct((E,), jnp.float32),
        ],
        scratch_types=(
            [pltpu.VMEM((NCHUNK, CH), jnp.int32)]
            + [pltpu.VMEM((CH, IN_C), jnp.float32) for _ in range(NBUF)]
            + [pltpu.VMEM((CH,), jnp.float32) for _ in range(3 * NBUF)]
            + [pltpu.SemaphoreType.DMA] * (2 * NBUF)
        ),
        compiler_params=pltpu.CompilerParams(use_tc_tiling_on_sc=False),
    )
    def gather_kernel(feat_hbm, sx_hbm, sy_hbm, sz_hbm, idx_hbm,
                      nf_hbm, xs_hbm, ys_hbm, zs_hbm, idx_all, *bufs):
        feat_v = bufs[:NBUF]
        pos_v = [bufs[NBUF + 3 * j:NBUF + 3 * (j + 1)] for j in range(NBUF)]
        gsem = bufs[4 * NBUF:5 * NBUF]
        wsem = bufs[5 * NBUF:]
        pos_hbms = (sx_hbm, sy_hbm, sz_hbm)
        out_hbms = (xs_hbm, ys_hbm, zs_hbm)
        wid = lax.axis_index("s") * NC + lax.axis_index("c")
        base = wid * PER_W

        pltpu.sync_copy(idx_hbm.at[wid], idx_all)

        def start_g(c, j):
            pltpu.async_copy(feat_hbm.at[idx_all.at[c]], feat_v[j], gsem[j])
            for t in range(3):
                pltpu.async_copy(pos_hbms[t].at[idx_all.at[c]],
                                 pos_v[j][t], gsem[j])

        def wait_g(j):
            pltpu.make_async_copy(feat_hbm.at[pl.ds(0, CH)], feat_v[j],
                                  gsem[j]).wait()
            for t in range(3):
                pltpu.make_async_copy(pos_hbms[t].at[pl.ds(0, CH)],
                                      pos_v[j][t], gsem[j]).wait()

        def start_w(c, j):
            off = base + c * CH
            pltpu.async_copy(feat_v[j], nf_hbm.at[pl.ds(off, CH)], wsem[j])
            for t in range(3):
                pltpu.async_copy(pos_v[j][t],
                                 out_hbms[t].at[pl.ds(off, CH)], wsem[j])

        def wait_w(j):
            pltpu.make_async_copy(feat_v[j], nf_hbm.at[pl.ds(0, CH)],
                                  wsem[j]).wait()
            for t in range(3):
                pltpu.make_async_copy(pos_v[j][t],
                                      out_hbms[t].at[pl.ds(0, CH)],
                                      wsem[j]).wait()

        for j in range(NBUF):
            start_g(j, j)

        def body(cc, carry):
            for j in range(NBUF):
                c = cc * NBUF + j
                wait_g(j)
                start_w(c, j)

                @pl.when(cc < NSUP - 1)
                def _():
                    wait_w(j)
                    start_g(c + NBUF, j)

            return carry

        lax.fori_loop(0, NSUP, body, 0)
        for j in range(NBUF):
            wait_w(j)

    return gather_kernel(features, supx, supy, supz, idx_grp)


def _tc_body(nf_ref, xs_ref, ys_ref, zs_ref, qx_ref, qy_ref, qz_ref,
             kpt_ref, kv_ref, out_ref, wf_ref):
    f32 = jnp.float32
    bf16 = jnp.bfloat16
    dx = (xs_ref[...] - qx_ref[...]).reshape(1, EPB)
    dy = (ys_ref[...] - qy_ref[...]).reshape(1, EPB)
    dz = (zs_ref[...] - qz_ref[...]).reshape(1, EPB)
    ddx = dx - kpt_ref[:, 0:1]
    ddy = dy - kpt_ref[:, 1:2]
    ddz = dz - kpt_ref[:, 2:3]
    sq = ddx * ddx + ddy * ddy + ddz * ddz
    w2 = jnp.maximum(1.0 - jnp.sqrt(sq) * (1.0 / KP_EXTENT), 0.0)

    ri = lax.broadcasted_iota(jnp.int32, (128, 2 * 128), 0)
    ci = lax.broadcasted_iota(jnp.int32, (128, 2 * 128), 1)
    mask = ((ri % 8) == (ci // NN)).astype(f32)

    for g in range(G):
        w_g = w2[:, 256 * g:256 * (g + 1)]
        bd = jnp.broadcast_to(w_g[:, None, :],
                              (16, 8, 256)).reshape(128, 256) * mask
        nf_g = nf_ref[pl.ds(256 * g, 256), :]
        wf_g = jnp.dot(bd, nf_g, preferred_element_type=f32)
        wf_ref[:, 8 * g:8 * (g + 1), :] = wf_g.astype(bf16).reshape(16, 8, 128)

    acc = jnp.zeros((B, OUT_C), f32)
    for k in range(16):
        acc = acc + jnp.dot(wf_ref[k], kv_ref[k], preferred_element_type=f32)
    out_ref[...] = acc


def kernel(query, support, edge_indices, features, K_points, K_values):
    f32 = jnp.float32
    padv = (jnp.arange(E - N * NN, dtype=jnp.int32) * 53) % N
    idx_pad = jnp.concatenate(
        [edge_indices.astype(jnp.int32).reshape(-1), padv])
    idx_grp = idx_pad.reshape(NW, NCHUNK, CH)
    sup = support.astype(f32)
    nf, xs, ys, zs = _sc_gather(features.astype(f32), sup[:, 0], sup[:, 1],
                                sup[:, 2], idx_grp)

    q = jnp.pad(query.astype(f32), ((0, NP - N), (0, 0)))
    qxr = jnp.repeat(q[:, 0], NN)
    qyr = jnp.repeat(q[:, 1], NN)
    qzr = jnp.repeat(q[:, 2], NN)
    kpt = jnp.pad(K_points.astype(f32), ((0, 1), (0, 1)),
                  constant_values=1e6)
    kv = jnp.pad(K_values.astype(jnp.bfloat16),
                 ((0, 1), (0, 0), (0, 0)))

    vec = pl.BlockSpec((EPB,), lambda i: (i,))
    out = pl.pallas_call(
        _tc_body,
        grid=(NBLK,),
        in_specs=[
            pl.BlockSpec((EPB, IN_C), lambda i: (i, 0)),
            vec, vec, vec, vec, vec, vec,
            pl.BlockSpec((16, 4), lambda i: (0, 0)),
            pl.BlockSpec((16, 128, 128), lambda i: (0, 0, 0)),
        ],
        out_specs=pl.BlockSpec((B, OUT_C), lambda i: (i, 0)),
        out_shape=jax.ShapeDtypeStruct((NP, OUT_C), f32),
        scratch_shapes=[pltpu.VMEM((16, B, 128), jnp.bfloat16)],
    )(nf, xs, ys, zs, qxr, qyr, qzr, kpt, kv)
    return out[:N]


# speedup vs baseline: 3.3508x; 1.1136x over previous
import functools

import jax
import jax.numpy as jnp
from jax import lax
from jax.experimental import pallas as pl
from jax.experimental.pallas import tpu as pltpu
from jax.experimental.pallas import tpu_sc as plsc

N = 10000
NN = 32
IN_C = 128
OUT_C = 128
K = 15
KP_EXTENT = 0.1

NP = 10240
E = NP * NN

B = 320
G = B // 8
EPB = B * NN
NBLK = NP // B

NC = 2
NS = 16
NW = NC * NS
PER_W = E // NW
CH = 128
NHALF = 2
EH = E // NHALF
PER_WH = EH // NW
NCHUNK = PER_WH // CH
NBUF = 5
NSUP = NCHUNK // NBUF


def _sc_gather(features, supx, supy, supz, idx_grp):
    mesh = plsc.VectorSubcoreMesh(core_axis_name="c", subcore_axis_name="s")

    @functools.partial(
        pl.kernel,
        mesh=mesh,
        out_type=[
            jax.ShapeDtypeStruct((EH, IN_C), jnp.float32),
            jax.ShapeDtypeStruct((EH,), jnp.float32),
            jax.ShapeDtypeStruct((EH,), jnp.float32),
            jax.ShapeDtypeStruct((EH,), jnp.float32),
        ],
        scratch_types=(
            [pltpu.VMEM((NCHUNK, CH), jnp.int32)]
            + [pltpu.VMEM((CH, IN_C), jnp.float32) for _ in range(NBUF)]
            + [pltpu.VMEM((CH,), jnp.float32) for _ in range(3 * NBUF)]
            + [pltpu.SemaphoreType.DMA] * (2 * NBUF)
        ),
        compiler_params=pltpu.CompilerParams(use_tc_tiling_on_sc=False),
    )
    def gather_kernel(feat_hbm, sx_hbm, sy_hbm, sz_hbm, idx_hbm,
                      nf_hbm, xs_hbm, ys_hbm, zs_hbm, idx_all, *bufs):
        feat_v = bufs[:NBUF]
        pos_v = [bufs[NBUF + 3 * j:NBUF + 3 * (j + 1)] for j in range(NBUF)]
        gsem = bufs[4 * NBUF:5 * NBUF]
        wsem = bufs[5 * NBUF:]
        pos_hbms = (sx_hbm, sy_hbm, sz_hbm)
        out_hbms = (xs_hbm, ys_hbm, zs_hbm)
        wid = lax.axis_index("s") * NC + lax.axis_index("c")
        base = wid * PER_WH

        pltpu.sync_copy(idx_hbm.at[wid], idx_all)

        def start_g(c, j):
            pltpu.async_copy(feat_hbm.at[idx_all.at[c]], feat_v[j], gsem[j])
            for t in range(3):
                pltpu.async_copy(pos_hbms[t].at[idx_all.at[c]],
                                 pos_v[j][t], gsem[j])

        def wait_g(j):
            pltpu.make_async_copy(feat_hbm.at[pl.ds(0, CH)], feat_v[j],
                                  gsem[j]).wait()
            for t in range(3):
                pltpu.make_async_copy(pos_hbms[t].at[pl.ds(0, CH)],
                                      pos_v[j][t], gsem[j]).wait()

        def start_w(c, j):
            off = base + c * CH
            pltpu.async_copy(feat_v[j], nf_hbm.at[pl.ds(off, CH)], wsem[j])
            for t in range(3):
                pltpu.async_copy(pos_v[j][t],
                                 out_hbms[t].at[pl.ds(off, CH)], wsem[j])

        def wait_w(j):
            pltpu.make_async_copy(feat_v[j], nf_hbm.at[pl.ds(0, CH)],
                                  wsem[j]).wait()
            for t in range(3):
                pltpu.make_async_copy(pos_v[j][t],
                                      out_hbms[t].at[pl.ds(0, CH)],
                                      wsem[j]).wait()

        for j in range(NBUF):
            start_g(j, j)

        def body(cc, carry):
            for j in range(NBUF):
                c = cc * NBUF + j
                wait_g(j)
                start_w(c, j)

                @pl.when(cc < NSUP - 1)
                def _():
                    wait_w(j)
                    start_g(c + NBUF, j)

            return carry

        lax.fori_loop(0, NSUP, body, 0)
        for j in range(NBUF):
            wait_w(j)

    return gather_kernel(features, supx, supy, supz, idx_grp)


def _tc_body(nf_ref, xs_ref, ys_ref, zs_ref, qx_ref, qy_ref, qz_ref,
             kpt_ref, kv_ref, out_ref, wf_ref):
    f32 = jnp.float32
    bf16 = jnp.bfloat16
    dx = (xs_ref[...] - qx_ref[...]).reshape(1, EPB)
    dy = (ys_ref[...] - qy_ref[...]).reshape(1, EPB)
    dz = (zs_ref[...] - qz_ref[...]).reshape(1, EPB)
    ddx = dx - kpt_ref[:, 0:1]
    ddy = dy - kpt_ref[:, 1:2]
    ddz = dz - kpt_ref[:, 2:3]
    sq = ddx * ddx + ddy * ddy + ddz * ddz
    w2 = jnp.maximum(1.0 - jnp.sqrt(sq) * (1.0 / KP_EXTENT), 0.0)

    ri = lax.broadcasted_iota(jnp.int32, (128, 2 * 128), 0)
    ci = lax.broadcasted_iota(jnp.int32, (128, 2 * 128), 1)
    mask = ((ri % 8) == (ci // NN)).astype(f32)

    for g in range(G):
        w_g = w2[:, 256 * g:256 * (g + 1)]
        bd = jnp.broadcast_to(w_g[:, None, :],
                              (16, 8, 256)).reshape(128, 256) * mask
        nf_g = nf_ref[pl.ds(256 * g, 256), :]
        wf_g = jnp.dot(bd, nf_g, preferred_element_type=f32)
        wf_ref[:, 8 * g:8 * (g + 1), :] = wf_g.astype(bf16).reshape(16, 8, 128)

    acc = jnp.zeros((B, OUT_C), f32)
    for k in range(16):
        acc = acc + jnp.dot(wf_ref[k], kv_ref[k], preferred_element_type=f32)
    out_ref[...] = acc


def kernel(query, support, edge_indices, features, K_points, K_values):
    f32 = jnp.float32
    padv = (jnp.arange(E - N * NN, dtype=jnp.int32) * 53) % N
    idx_pad = jnp.concatenate(
        [edge_indices.astype(jnp.int32).reshape(-1), padv])
    sup = support.astype(f32)
    feat = features.astype(f32)

    q = jnp.pad(query.astype(f32), ((0, NP - N), (0, 0)))
    qxr = jnp.repeat(q[:, 0], NN)
    qyr = jnp.repeat(q[:, 1], NN)
    qzr = jnp.repeat(q[:, 2], NN)
    kpt = jnp.pad(K_points.astype(f32), ((0, 1), (0, 1)),
                  constant_values=1e6)
    kv = jnp.pad(K_values.astype(jnp.bfloat16),
                 ((0, 1), (0, 0), (0, 0)))

    vec = pl.BlockSpec((EPB,), lambda i: (i,))
    tc = pl.pallas_call(
        _tc_body,
        grid=(NBLK // NHALF,),
        in_specs=[
            pl.BlockSpec((EPB, IN_C), lambda i: (i, 0)),
            vec, vec, vec, vec, vec, vec,
            pl.BlockSpec((16, 4), lambda i: (0, 0)),
            pl.BlockSpec((16, 128, 128), lambda i: (0, 0, 0)),
        ],
        out_specs=pl.BlockSpec((B, OUT_C), lambda i: (i, 0)),
        out_shape=jax.ShapeDtypeStruct((NP // NHALF, OUT_C), f32),
        scratch_shapes=[pltpu.VMEM((16, B, 128), jnp.bfloat16)],
    )

    gathered = []
    for h in range(NHALF):
        idx_h = idx_pad[h * EH:(h + 1) * EH].reshape(NW, NCHUNK, CH)
        gathered.append(_sc_gather(feat, sup[:, 0], sup[:, 1], sup[:, 2],
                                   idx_h))
    outs = []
    for h, (nf, xs, ys, zs) in enumerate(gathered):
        outs.append(tc(nf, xs, ys, zs, qxr[h * EH:(h + 1) * EH],
                       qyr[h * EH:(h + 1) * EH], qzr[h * EH:(h + 1) * EH],
                       kpt, kv))
    return jnp.concatenate(outs)[:N]
